# baseline (speedup 1.0000x reference)
import math

import jax
import jax.numpy as jnp
from jax import lax
from jax.experimental import pallas as pl
from jax.experimental.pallas import tpu as pltpu

_BIG = 1e8
_LOG2E = math.log2(math.e)
_LN2 = math.log(2.0)


def _dtw_kernel(x_ref, y_ref, out_ref, s0_ref, s1_ref,
                r1a_ref, ru1a_ref, ru2a_ref, r1b_ref, ru1b_ref, ru2b_ref):
    BB, N, D = x_ref.shape
    f32 = jnp.float32
    big = f32(_BIG)

    def mm_body(b, carry):
        xb = x_ref[b]
        yb = y_ref[b]
        x2 = jnp.sum(xb * xb, axis=1, keepdims=True)
        y2 = jnp.sum(yb * yb, axis=1, keepdims=True)
        ones = jnp.ones((N, 1), dtype=f32)
        xh = jnp.concatenate([xb, x2, ones], axis=1)
        yh = jnp.concatenate([-2.0 * yb, ones, y2], axis=1)
        nt = N // 4
        for t in range(4):
            s0_ref[b, t * nt:(t + 1) * nt, :] = lax.dot_general(
                yh[t * nt:(t + 1) * nt, :], xh,
                (((1,), (1,)), ((), ())),
                preferred_element_type=f32)
        return carry

    lax.fori_loop(0, BB, mm_body, 0)

    nbits = N.bit_length() - 1
    lane = lax.broadcasted_iota(jnp.int32, (1, N), 1)
    bufs = [s0_ref, s1_ref]
    for k in range(nbits):
        src = bufs[k % 2]
        dst = bufs[(k + 1) % 2]
        s = 1 << k
        mask = ((lane >> k) & 1) == 1

        def sh_body(b, carry):
            cur = src[b]
            rolled = jnp.concatenate([cur[N - s:, :], cur[:N - s, :]],
                                     axis=0)
            dst[b] = jnp.where(mask, rolled, cur)
            return carry

        lax.fori_loop(0, BB, sh_body, 0)
    sfin = bufs[nbits % 2]

    H = BB // 2
    av = lax.broadcasted_iota(jnp.int32, (H, N), 1)
    lane0 = av == 0
    state = [(r1a_ref, ru1a_ref, ru2a_ref), (r1b_ref, ru1b_ref, ru2b_ref)]
    for st in state:
        st[0][...] = jnp.full((H, N), big, dtype=f32)
        st[1][...] = jnp.full((H, N), big, dtype=f32)
        st[2][...] = jnp.where(lane0, f32(0.0), big)

    def step(e, cost, r1, ru1, ru2, phase):
        m = jnp.minimum(jnp.minimum(ru1, ru2), r1)
        ssum = jnp.exp(m - ru1) + jnp.exp(m - r1) + jnp.exp(m - ru2)
        smin = m - jnp.log(ssum)
        if phase == 0:
            valid = av <= e
        else:
            valid = av >= e - (N - 1)
        new = jnp.where(valid, cost + smin, big)
        ru_new = jnp.where(
            lane0, big,
            jnp.concatenate([new[:, N - 1:], new[:, :N - 1]], axis=1))
        return new, ru_new, ru1

    def make_body(phase):
        def body(q, carry):
            c0 = pl.multiple_of((q * 8) & (N - 1), 8)
            for h in range(2):
                bs, be = h * H, (h + 1) * H
                r1_ref, ru1_ref, ru2_ref = state[h]
                slab = sfin[bs:be, pl.ds(c0, 8), :]
                r1, ru1, ru2 = r1_ref[...], ru1_ref[...], ru2_ref[...]
                for kk in range(8):
                    r1, ru1, ru2 = step(q * 8 + kk, slab[:, kk, :],
                                        r1, ru1, ru2, phase)
                r1_ref[...] = r1
                ru1_ref[...] = ru1
                ru2_ref[...] = ru2
            return carry

        return body

    nq = (2 * N) // 8
    lax.fori_loop(0, nq // 2, make_body(0), 0)
    lax.fori_loop(nq // 2, nq - 1, make_body(1), 0)
    for h in range(2):
        bs, be = h * H, (h + 1) * H
        r1_ref, ru1_ref, ru2_ref = state[h]
        slab = sfin[bs:be, N - 8:N, :]
        r1, ru1, ru2 = r1_ref[...], ru1_ref[...], ru2_ref[...]
        for kk in range(7):
            r1, ru1, ru2 = step(2 * N - 8 + kk, slab[:, kk, :],
                                r1, ru1, ru2, 1)
        out_ref[bs:be, :] = r1[:, N - 1:N]


def _dtw_batch(x, y, bb, interpret=False):
    B, N, D = x.shape
    grid = (B // bb,)
    return pl.pallas_call(
        _dtw_kernel,
        out_shape=jax.ShapeDtypeStruct((B, 1), jnp.float32),
        grid=grid,
        in_specs=[
            pl.BlockSpec((bb, N, D), lambda p: (p, 0, 0)),
            pl.BlockSpec((bb, N, D), lambda p: (p, 0, 0)),
        ],
        out_specs=pl.BlockSpec((bb, 1), lambda p: (p, 0)),
        scratch_shapes=[
            pltpu.VMEM((bb, N, N), jnp.float32),
            pltpu.VMEM((bb, N, N), jnp.float32),
        ] + [pltpu.VMEM((bb // 2, N), jnp.float32) for _ in range(6)],
        compiler_params=pltpu.CompilerParams(
            dimension_semantics=("parallel",),
            vmem_limit_bytes=50 * 1024 * 1024,
        ),
        name="soft_dtw",
        interpret=interpret,
    )(x, y)


def kernel(inputs, targets):
    r = _dtw_batch(inputs, targets, bb=16)
    return jnp.mean(r)


# speedup vs baseline: 1.0872x; 1.0872x over previous
import math

import jax
import jax.numpy as jnp
from jax import lax
from jax.experimental import pallas as pl
from jax.experimental.pallas import tpu as pltpu

_BIG = 1e8
_LOG2E = math.log2(math.e)
_LN2 = math.log(2.0)


def _dtw_kernel(x_ref, y_ref, out_ref, s0_ref, s1_ref, rows_ref,
                r1a_ref, ru1a_ref, ru2a_ref, r1b_ref, ru1b_ref, ru2b_ref):
    BB, N, D = x_ref.shape
    f32 = jnp.float32
    big = f32(_BIG)

    def mm_body(b, carry):
        xb = x_ref[b]
        yb = y_ref[b]
        x2 = jnp.sum(xb * xb, axis=1, keepdims=True)
        y2 = jnp.sum(yb * yb, axis=1, keepdims=True)
        ones = jnp.ones((N, 1), dtype=f32)
        xh = jnp.concatenate([xb, x2, ones], axis=1)
        yh = jnp.concatenate([-2.0 * yb, ones, y2], axis=1)
        nt = N // 4
        for t in range(4):
            s0_ref[b, t * nt:(t + 1) * nt, :] = lax.dot_general(
                yh[t * nt:(t + 1) * nt, :], xh,
                (((1,), (1,)), ((), ())),
                preferred_element_type=f32)
        return carry

    lax.fori_loop(0, BB, mm_body, 0)

    nbits = N.bit_length() - 1
    lane = lax.broadcasted_iota(jnp.int32, (1, N), 1)
    bufs = [s0_ref, s1_ref]
    for k in range(nbits):
        src = bufs[k % 2]
        dst = bufs[(k + 1) % 2]
        s = 1 << k
        mask = ((lane >> k) & 1) == 1

        def sh_body(b, carry):
            cur = src[b]
            rolled = jnp.concatenate([cur[N - s:, :], cur[:N - s, :]],
                                     axis=0)
            dst[b] = jnp.where(mask, rolled, cur)
            return carry

        lax.fori_loop(0, BB, sh_body, 0)
    sfin = bufs[nbits % 2]

    H = BB // 2
    av = lax.broadcasted_iota(jnp.int32, (H, N), 1)
    lane0 = av == 0
    state = [(r1a_ref, ru1a_ref, ru2a_ref), (r1b_ref, ru1b_ref, ru2b_ref)]
    for st in state:
        st[0][...] = jnp.full((H, N), big, dtype=f32)
        st[1][...] = jnp.full((H, N), big, dtype=f32)
        st[2][...] = jnp.where(lane0, f32(0.0), big)

    def step(e, cost, r1, ru1, ru2, phase):
        m = jnp.minimum(jnp.minimum(ru1, ru2), r1)
        ssum = jnp.exp(m - ru1) + jnp.exp(m - r1) + jnp.exp(m - ru2)
        smin = m - jnp.log(ssum)
        if phase == 0:
            valid = av <= e
        else:
            valid = av >= e - (N - 1)
        new = jnp.where(valid, cost + smin, big)
        ru_new = jnp.where(
            lane0, big,
            jnp.concatenate([new[:, N - 1:], new[:, :N - 1]], axis=1))
        return new, ru_new, ru1

    def assemble(c0, slot):
        slab = sfin[:, pl.ds(c0, 8), :]
        for kk in range(8):
            rows_ref[slot, kk] = slab[:, kk, :]

    assemble(0, 0)

    def make_body(phase):
        def body(q, carry):
            slot = lax.rem(q, 2)
            c0n = pl.multiple_of(((q + 1) * 8) & (N - 1), 8)
            assemble(c0n, 1 - slot)
            for h in range(2):
                bs, be = h * H, (h + 1) * H
                r1_ref, ru1_ref, ru2_ref = state[h]
                r1, ru1, ru2 = r1_ref[...], ru1_ref[...], ru2_ref[...]
                for kk in range(8):
                    r1, ru1, ru2 = step(q * 8 + kk,
                                        rows_ref[slot, kk, bs:be, :],
                                        r1, ru1, ru2, phase)
                r1_ref[...] = r1
                ru1_ref[...] = ru1
                ru2_ref[...] = ru2
            return carry

        return body

    nq = (2 * N) // 8
    lax.fori_loop(0, nq // 2, make_body(0), 0)
    lax.fori_loop(nq // 2, nq - 1, make_body(1), 0)
    for h in range(2):
        bs, be = h * H, (h + 1) * H
        r1_ref, ru1_ref, ru2_ref = state[h]
        r1, ru1, ru2 = r1_ref[...], ru1_ref[...], ru2_ref[...]
        for kk in range(7):
            r1, ru1, ru2 = step(2 * N - 8 + kk,
                                rows_ref[1, kk, bs:be, :],
                                r1, ru1, ru2, 1)
        out_ref[bs:be, :] = r1[:, N - 1:N]


def _dtw_batch(x, y, bb, interpret=False):
    B, N, D = x.shape
    grid = (B // bb,)
    return pl.pallas_call(
        _dtw_kernel,
        out_shape=jax.ShapeDtypeStruct((B, 1), jnp.float32),
        grid=grid,
        in_specs=[
            pl.BlockSpec((bb, N, D), lambda p: (p, 0, 0)),
            pl.BlockSpec((bb, N, D), lambda p: (p, 0, 0)),
        ],
        out_specs=pl.BlockSpec((bb, 1), lambda p: (p, 0)),
        scratch_shapes=[
            pltpu.VMEM((bb, N, N), jnp.float32),
            pltpu.VMEM((bb, N, N), jnp.float32),
            pltpu.VMEM((2, 8, bb, N), jnp.float32),
        ] + [pltpu.VMEM((bb // 2, N), jnp.float32) for _ in range(6)],
        compiler_params=pltpu.CompilerParams(
            dimension_semantics=("parallel",),
            vmem_limit_bytes=50 * 1024 * 1024,
        ),
        name="soft_dtw",
        interpret=interpret,
    )(x, y)


def kernel(inputs, targets):
    r = _dtw_batch(inputs, targets, bb=16)
    return jnp.mean(r)


# speedup vs baseline: 1.0922x; 1.0046x over previous
import math

import jax
import jax.numpy as jnp
from jax import lax
from jax.experimental import pallas as pl
from jax.experimental.pallas import tpu as pltpu

_BIG = 1e8
_LOG2E = math.log2(math.e)
_LN2 = math.log(2.0)


def _dtw_kernel(x_ref, y_ref, out_ref, s0_ref, s1_ref, rows_ref,
                r1a_ref, ru1a_ref, ru2a_ref, r1b_ref, ru1b_ref, ru2b_ref):
    BB, N, D = x_ref.shape
    f32 = jnp.float32
    big = f32(_BIG)

    def mm_body(b, carry):
        xb = x_ref[b]
        yb = y_ref[b]
        x2 = jnp.sum(xb * xb, axis=1, keepdims=True)
        y2 = jnp.sum(yb * yb, axis=1, keepdims=True)
        ones = jnp.ones((N, 1), dtype=f32)
        xh = jnp.concatenate([xb, x2, ones], axis=1)
        yh = jnp.concatenate([-2.0 * yb, ones, y2], axis=1)
        nt = N // 4
        for t in range(4):
            s0_ref[b, t * nt:(t + 1) * nt, :] = lax.dot_general(
                yh[t * nt:(t + 1) * nt, :], xh,
                (((1,), (1,)), ((), ())),
                preferred_element_type=f32)
        return carry

    lax.fori_loop(0, BB, mm_body, 0)

    nbits = N.bit_length() - 1
    lane = lax.broadcasted_iota(jnp.int32, (1, N), 1)
    bufs = [s0_ref, s1_ref]
    for k in range(nbits):
        src = bufs[k % 2]
        dst = bufs[(k + 1) % 2]
        s = 1 << k
        mask = ((lane >> k) & 1) == 1

        def sh_body(b, carry):
            cur = src[b]
            rolled = jnp.concatenate([cur[N - s:, :], cur[:N - s, :]],
                                     axis=0)
            dst[b] = jnp.where(mask, rolled, cur)
            return carry

        lax.fori_loop(0, BB, sh_body, 0)
    sfin = bufs[nbits % 2]

    H = BB
    av = lax.broadcasted_iota(jnp.int32, (H, N), 1)
    lane0 = av == 0
    state = [(r1a_ref, ru1a_ref, ru2a_ref)]
    for st in state:
        st[0][...] = jnp.full((H, N), big, dtype=f32)
        st[1][...] = jnp.full((H, N), big, dtype=f32)
        st[2][...] = jnp.where(lane0, f32(0.0), big)

    def step(e, cost, r1, ru1, ru2, phase):
        m = jnp.minimum(jnp.minimum(ru1, ru2), r1)
        ssum = jnp.exp(m - ru1) + jnp.exp(m - r1) + jnp.exp(m - ru2)
        smin = m - jnp.log(ssum)
        if phase == 0:
            valid = av <= e
        else:
            valid = av >= e - (N - 1)
        new = jnp.where(valid, cost + smin, big)
        ru_new = jnp.where(
            lane0, big,
            jnp.concatenate([new[:, N - 1:], new[:, :N - 1]], axis=1))
        return new, ru_new, ru1

    def assemble(c0, slot):
        slab = sfin[:, pl.ds(c0, 8), :]
        for kk in range(8):
            rows_ref[slot, kk] = slab[:, kk, :]

    assemble(0, 0)

    def make_body(phase):
        def body(q, carry):
            slot = lax.rem(q, 2)
            c0n = pl.multiple_of(((q + 1) * 8) & (N - 1), 8)
            assemble(c0n, 1 - slot)
            for h in range(1):
                r1_ref, ru1_ref, ru2_ref = state[h]
                r1, ru1, ru2 = r1_ref[...], ru1_ref[...], ru2_ref[...]
                for kk in range(8):
                    r1, ru1, ru2 = step(q * 8 + kk,
                                        rows_ref[slot, kk],
                                        r1, ru1, ru2, phase)
                r1_ref[...] = r1
                ru1_ref[...] = ru1
                ru2_ref[...] = ru2
            return carry

        return body

    nq = (2 * N) // 8
    lax.fori_loop(0, nq // 2, make_body(0), 0)
    lax.fori_loop(nq // 2, nq - 1, make_body(1), 0)
    for h in range(1):
        r1_ref, ru1_ref, ru2_ref = state[h]
        r1, ru1, ru2 = r1_ref[...], ru1_ref[...], ru2_ref[...]
        for kk in range(7):
            r1, ru1, ru2 = step(2 * N - 8 + kk,
                                rows_ref[1, kk],
                                r1, ru1, ru2, 1)
        out_ref[...] = r1[:, N - 1:N]


def _dtw_batch(x, y, bb, interpret=False):
    B, N, D = x.shape
    grid = (B // bb,)
    return pl.pallas_call(
        _dtw_kernel,
        out_shape=jax.ShapeDtypeStruct((B, 1), jnp.float32),
        grid=grid,
        in_specs=[
            pl.BlockSpec((bb, N, D), lambda p: (p, 0, 0)),
            pl.BlockSpec((bb, N, D), lambda p: (p, 0, 0)),
        ],
        out_specs=pl.BlockSpec((bb, 1), lambda p: (p, 0)),
        scratch_shapes=[
            pltpu.VMEM((bb, N, N), jnp.float32),
            pltpu.VMEM((bb, N, N), jnp.float32),
            pltpu.VMEM((2, 8, bb, N), jnp.float32),
        ] + [pltpu.VMEM((bb, N), jnp.float32) for _ in range(6)],
        compiler_params=pltpu.CompilerParams(
            dimension_semantics=("parallel",),
            vmem_limit_bytes=50 * 1024 * 1024,
        ),
        name="soft_dtw",
        interpret=interpret,
    )(x, y)


def kernel(inputs, targets):
    r = _dtw_batch(inputs, targets, bb=16)
    return jnp.mean(r)
